# baseline (speedup 1.0000x reference)
import functools

import jax
import jax.numpy as jnp
from jax import lax
from jax.experimental import pallas as pl
from jax.experimental.pallas import tpu as pltpu
from jax.experimental.pallas import tpu_sc as plsc

NC = 2
NS = 16
CHUNK = 112


def _make_deg_kernel(n_pad, n_chunks):
    rows_per_tile = n_pad // NS
    chunks_per_tile = n_chunks // (NC * NS)
    mesh = plsc.VectorSubcoreMesh(core_axis_name="c", subcore_axis_name="s")

    @functools.partial(
        pl.kernel,
        out_type=jax.ShapeDtypeStruct((NC * n_pad, 16), jnp.float32),
        mesh=mesh,
        scratch_types=[
            pltpu.VMEM_SHARED((n_pad, 16), jnp.float32),
            pltpu.VMEM((rows_per_tile, 16), jnp.float32),
            pltpu.VMEM((CHUNK, 16), jnp.float32),
            pltpu.VMEM((2, CHUNK), jnp.int32),
            pltpu.SemaphoreType.DMA,
        ],
    )
    def deg_kernel(edges_hbm, out_hbm, hist_sh, zbuf, ones_v, idx_v, isem):
        cid = lax.axis_index("c")
        sid = lax.axis_index("s")
        wid = sid * NC + cid
        rbase = sid * rows_per_tile

        zrow = jnp.zeros((16,), jnp.float32)
        orow = jnp.ones((16,), jnp.float32)

        def fill_z(i, _):
            zbuf[i, :] = zrow
            return ()
        lax.fori_loop(0, rows_per_tile, fill_z, ())

        def fill_o(i, _):
            ones_v[i, :] = orow
            return ()
        lax.fori_loop(0, CHUNK, fill_o, ())

        pltpu.sync_copy(zbuf, hist_sh.at[pl.ds(rbase, rows_per_tile)])
        plsc.subcore_barrier()

        cpt = chunks_per_tile
        cbase = wid * cpt
        pltpu.sync_copy(edges_hbm.at[cbase, 1], idx_v.at[0])
        pltpu.async_copy(edges_hbm.at[cbase + 1, 1], idx_v.at[1], isem)

        def outer(ci, _):
            for p in range(2):
                c = ci * 2 + p
                @pl.when(c + 1 < cpt)
                def _():
                    pltpu.make_async_copy(edges_hbm.at[cbase, 1],
                                          idx_v.at[1 - p], isem).wait()
                pltpu.sync_copy(ones_v, hist_sh.at[idx_v.at[p]], add=True)
                @pl.when(c + 2 < cpt)
                def _():
                    pltpu.async_copy(edges_hbm.at[cbase + c + 2, 1],
                                     idx_v.at[p], isem)
            return ()
        lax.fori_loop(0, cpt // 2, outer, ())

        plsc.subcore_barrier()
        pltpu.sync_copy(hist_sh.at[pl.ds(rbase, rows_per_tile)],
                        out_hbm.at[pl.ds(cid * n_pad + rbase, rows_per_tile)])

    return deg_kernel


def _make_spmm_kernel(n_pad, d, n_chunks, frac_a):
    rows_per_tile = n_pad // NS
    cpt_pair = n_chunks // NS
    cpt_a = int(round(cpt_pair * frac_a / 2)) * 2
    cpt_b = cpt_pair - cpt_a
    mesh = plsc.VectorSubcoreMesh(core_axis_name="c", subcore_axis_name="s")

    @functools.partial(
        pl.kernel,
        out_type=jax.ShapeDtypeStruct((NC * n_pad, d), jnp.float32),
        mesh=mesh,
        scratch_types=[
            pltpu.VMEM_SHARED((n_pad, d), jnp.float32),
            pltpu.VMEM((2, CHUNK, d), jnp.float32),
            pltpu.VMEM((2, 2, CHUNK), jnp.int32),
            pltpu.SemaphoreType.DMA((2,)),
            pltpu.SemaphoreType.DMA,
        ],
    )
    def spmm_kernel(h_hbm, edges_hbm, z_hbm, out_hbm,
                    acc_sh, rows_v, idx_v, gsem, isem):
        cid = lax.axis_index("c")
        sid = lax.axis_index("s")
        rbase = sid * rows_per_tile
        cpt = jnp.where(cid == 0, cpt_a, cpt_b)

        @pl.when(cid == 0)
        def _():
            pltpu.sync_copy(h_hbm.at[pl.ds(rbase, rows_per_tile)],
                            acc_sh.at[pl.ds(rbase, rows_per_tile)])

        @pl.when(cid == 1)
        def _():
            pltpu.sync_copy(z_hbm.at[pl.ds(rbase, rows_per_tile)],
                            acc_sh.at[pl.ds(rbase, rows_per_tile)])

        plsc.subcore_barrier()

        cbase = sid * cpt_pair + jnp.where(cid == 0, 0, cpt_a)
        pltpu.sync_copy(edges_hbm.at[cbase], idx_v.at[0])
        pltpu.async_copy(h_hbm.at[idx_v.at[0, 0]], rows_v.at[0], gsem.at[0])
        pltpu.async_copy(edges_hbm.at[cbase + 1], idx_v.at[1], isem)

        def outer(ci, _):
            for p in range(2):
                c = ci * 2 + p
                @pl.when(c + 1 < cpt)
                def _():
                    pltpu.make_async_copy(edges_hbm.at[cbase],
                                          idx_v.at[1 - p], isem).wait()
                    pltpu.async_copy(h_hbm.at[idx_v.at[1 - p, 0]],
                                     rows_v.at[1 - p], gsem.at[1 - p])
                pltpu.make_async_copy(h_hbm.at[idx_v.at[p, 0]],
                                      rows_v.at[p], gsem.at[p]).wait()
                pltpu.sync_copy(rows_v.at[p], acc_sh.at[idx_v.at[p, 1]],
                                add=True)
                @pl.when(c + 2 < cpt)
                def _():
                    pltpu.async_copy(edges_hbm.at[cbase + c + 2],
                                     idx_v.at[p], isem)
            return ()
        lax.fori_loop(0, cpt // 2, outer, ())

        plsc.subcore_barrier()
        pltpu.sync_copy(acc_sh.at[pl.ds(rbase, rows_per_tile)],
                        out_hbm.at[pl.ds(cid * n_pad + rbase, rows_per_tile)])

    return spmm_kernel


def _tc_matmul(x_pad, w1, n_pad, d):
    blk = 256
    grid = n_pad // blk

    def body(x_ref, w_ref, out_ref):
        out_ref[...] = jnp.dot(x_ref[...], w_ref[...],
                               preferred_element_type=jnp.float32)

    return pl.pallas_call(
        body,
        grid=(grid,),
        in_specs=[
            pl.BlockSpec((blk, d), lambda i: (i, 0)),
            pl.BlockSpec((d, d), lambda i: (0, 0)),
        ],
        out_specs=pl.BlockSpec((blk, d), lambda i: (i, 0)),
        out_shape=jax.ShapeDtypeStruct((n_pad, d), jnp.float32),
    )(x_pad, w1)


def _tc_first(hist3, xw, n_pad, d):
    blk = 256
    grid = n_pad // blk

    def body(hist_ref, xw_ref, h_ref, dis_ref):
        deg = hist_ref[0, :, 0:1] + hist_ref[1, :, 0:1] + 1.0
        dis = lax.rsqrt(deg)
        h_ref[...] = xw_ref[...] * dis
        dis_ref[...] = jnp.broadcast_to(dis, (blk, d))

    return pl.pallas_call(
        body,
        grid=(grid,),
        in_specs=[
            pl.BlockSpec((2, blk, 16), lambda i: (0, i, 0)),
            pl.BlockSpec((blk, d), lambda i: (i, 0)),
        ],
        out_specs=[
            pl.BlockSpec((blk, d), lambda i: (i, 0)),
            pl.BlockSpec((blk, d), lambda i: (i, 0)),
        ],
        out_shape=[
            jax.ShapeDtypeStruct((n_pad, d), jnp.float32),
            jax.ShapeDtypeStruct((n_pad, d), jnp.float32),
        ],
    )(hist3, xw)


def _tc_mid(acc, dis_col, b1r, w2, n_pad, d):
    blk = 256
    grid = n_pad // blk

    def body(a0_ref, a1_ref, dis_ref, b_ref, w_ref, out_ref):
        dis = dis_ref[...]
        agg = (a0_ref[...] + a1_ref[...]) * dis
        h = jnp.maximum(agg + b_ref[...], 0.0)
        hw = jnp.dot(h, w_ref[...], preferred_element_type=jnp.float32)
        out_ref[...] = hw * dis

    return pl.pallas_call(
        body,
        grid=(grid,),
        in_specs=[
            pl.BlockSpec((blk, d), lambda i: (i, 0)),
            pl.BlockSpec((blk, d), lambda i: (i + grid, 0)),
            pl.BlockSpec((blk, d), lambda i: (i, 0)),
            pl.BlockSpec((1, d), lambda i: (0, 0)),
            pl.BlockSpec((d, d), lambda i: (0, 0)),
        ],
        out_specs=pl.BlockSpec((blk, d), lambda i: (i, 0)),
        out_shape=jax.ShapeDtypeStruct((n_pad, d), jnp.float32),
    )(acc, acc, dis_col, b1r, w2)


def _tc_last(acc, dis_col, b2r, n, n_pad, d):
    blk = 400
    grid = n // blk

    def body(a0_ref, a1_ref, dis_ref, b_ref, out_ref):
        z = (a0_ref[...] + a1_ref[...]) * dis_ref[...] + b_ref[...]
        m = jnp.max(z, axis=1, keepdims=True)
        ex = jnp.exp(z - m)
        s = jnp.sum(ex, axis=1, keepdims=True)
        out_ref[...] = z - m - jnp.log(s)

    nblk_off = n_pad // blk

    return pl.pallas_call(
        body,
        grid=(grid,),
        in_specs=[
            pl.BlockSpec((blk, d), lambda i: (i, 0)),
            pl.BlockSpec((blk, d), lambda i: (i + nblk_off, 0)),
            pl.BlockSpec((blk, d), lambda i: (i, 0)),
            pl.BlockSpec((1, d), lambda i: (0, 0)),
        ],
        out_specs=pl.BlockSpec((blk, d), lambda i: (i, 0)),
        out_shape=jax.ShapeDtypeStruct((n, d), jnp.float32),
    )(acc, acc, dis_col, b2r)


def kernel(x, edge_index, W1, b1, W2, b2):
    n, d = x.shape
    e = edge_index.shape[1]

    n_pad = ((n + NS * 256 - 1) // (NS * 256)) * NS * 256
    pad_row = n_pad - 1
    chunks_per_tile = -(-e // (NC * NS * CHUNK))
    chunks_per_tile = ((chunks_per_tile + 3) // 4) * 4
    n_chunks = chunks_per_tile * NC * NS
    e_pad = n_chunks * CHUNK

    src = edge_index[0].astype(jnp.int32)
    dst = edge_index[1].astype(jnp.int32)
    pad = jnp.full((e_pad - e,), pad_row, jnp.int32)
    edges = jnp.stack([
        jnp.concatenate([src, pad]).reshape(n_chunks, CHUNK),
        jnp.concatenate([dst, pad]).reshape(n_chunks, CHUNK),
    ], axis=1)

    x_pad = jnp.concatenate(
        [x, jnp.zeros((n_pad - n, d), jnp.float32)], axis=0)
    zinit = jnp.zeros((n_pad, d), jnp.float32)
    b1r = b1.reshape(1, d)
    b2r = b2.reshape(1, d)

    deg_k = _make_deg_kernel(n_pad, n_chunks)
    spmm_k = _make_spmm_kernel(n_pad, d, n_chunks, 132.0 / 180.0)

    xw = _tc_matmul(x_pad, W1, n_pad, d)
    hist = deg_k(edges).reshape(NC, n_pad, 16)
    h1, dis_col = _tc_first(hist, xw, n_pad, d)
    acc1 = spmm_k(h1, edges, zinit)
    h2 = _tc_mid(acc1, dis_col, b1r, W2, n_pad, d)
    acc2 = spmm_k(h2, edges, zinit)
    return _tc_last(acc2, dis_col, b2r, n, n_pad, d)


# speedup vs baseline: 1.9306x; 1.9306x over previous
import functools

import jax
import jax.numpy as jnp
from jax import lax
from jax.experimental import pallas as pl
from jax.experimental.pallas import tpu as pltpu
from jax.experimental.pallas import tpu_sc as plsc

NC = 2
NS = 16
CHUNK = 112


def _make_deg_kernel(n_pad, n_chunks):
    rows_per_tile = n_pad // NS
    chunks_per_tile = n_chunks // (NC * NS)
    mesh = plsc.VectorSubcoreMesh(core_axis_name="c", subcore_axis_name="s")

    @functools.partial(
        pl.kernel,
        out_type=jax.ShapeDtypeStruct((NC * n_pad, 16), jnp.float32),
        mesh=mesh,
        scratch_types=[
            pltpu.VMEM_SHARED((n_pad, 16), jnp.float32),
            pltpu.VMEM((rows_per_tile, 16), jnp.float32),
            pltpu.VMEM((CHUNK, 16), jnp.float32),
            pltpu.VMEM((2, CHUNK), jnp.int32),
            pltpu.SemaphoreType.DMA,
        ],
    )
    def deg_kernel(edges_hbm, out_hbm, hist_sh, zbuf, ones_v, idx_v, isem):
        cid = lax.axis_index("c")
        sid = lax.axis_index("s")
        wid = sid * NC + cid
        rbase = sid * rows_per_tile

        zrow = jnp.zeros((16,), jnp.float32)
        orow = jnp.ones((16,), jnp.float32)

        def fill_z(i, _):
            zbuf[i, :] = zrow
            return ()
        lax.fori_loop(0, rows_per_tile, fill_z, ())

        def fill_o(i, _):
            ones_v[i, :] = orow
            return ()
        lax.fori_loop(0, CHUNK, fill_o, ())

        pltpu.sync_copy(zbuf, hist_sh.at[pl.ds(rbase, rows_per_tile)])
        plsc.subcore_barrier()

        cpt = chunks_per_tile
        cbase = wid * cpt
        pltpu.sync_copy(edges_hbm.at[cbase, 1], idx_v.at[0])
        pltpu.async_copy(edges_hbm.at[cbase + 1, 1], idx_v.at[1], isem)

        def outer(ci, _):
            for p in range(2):
                c = ci * 2 + p
                @pl.when(c + 1 < cpt)
                def _():
                    pltpu.make_async_copy(edges_hbm.at[cbase, 1],
                                          idx_v.at[1 - p], isem).wait()
                pltpu.sync_copy(ones_v, hist_sh.at[idx_v.at[p]], add=True)
                @pl.when(c + 2 < cpt)
                def _():
                    pltpu.async_copy(edges_hbm.at[cbase + c + 2, 1],
                                     idx_v.at[p], isem)
            return ()
        lax.fori_loop(0, cpt // 2, outer, ())

        plsc.subcore_barrier()
        pltpu.sync_copy(hist_sh.at[pl.ds(rbase, rows_per_tile)],
                        out_hbm.at[pl.ds(cid * n_pad + rbase, rows_per_tile)])

    return deg_kernel


def _make_spmm_kernel(n_pad, d, n_chunks, frac_a):
    rows_per_tile = n_pad // NS
    cpt_pair = n_chunks // NS
    cpt_a = int(round(cpt_pair * frac_a / 2)) * 2
    cpt_b = cpt_pair - cpt_a
    mesh = plsc.VectorSubcoreMesh(core_axis_name="c", subcore_axis_name="s")

    @functools.partial(
        pl.kernel,
        out_type=jax.ShapeDtypeStruct((NC * n_pad, d), jnp.float32),
        mesh=mesh,
        scratch_types=[
            pltpu.VMEM_SHARED((n_pad, d), jnp.float32),
            pltpu.VMEM((2, CHUNK, d), jnp.float32),
            pltpu.VMEM((2, 2, CHUNK), jnp.int32),
            pltpu.SemaphoreType.DMA((2,)),
            pltpu.SemaphoreType.DMA,
        ],
    )
    def spmm_kernel(h_hbm, edges_hbm, z_hbm, out_hbm,
                    acc_sh, rows_v, idx_v, gsem, isem):
        cid = lax.axis_index("c")
        sid = lax.axis_index("s")
        rbase = sid * rows_per_tile
        cpt = jnp.where(cid == 0, cpt_a, cpt_b)

        @pl.when(cid == 0)
        def _():
            pltpu.sync_copy(h_hbm.at[pl.ds(rbase, rows_per_tile)],
                            acc_sh.at[pl.ds(rbase, rows_per_tile)])

        @pl.when(cid == 1)
        def _():
            pltpu.sync_copy(z_hbm.at[pl.ds(rbase, rows_per_tile)],
                            acc_sh.at[pl.ds(rbase, rows_per_tile)])

        plsc.subcore_barrier()

        cbase = sid * cpt_pair + jnp.where(cid == 0, 0, cpt_a)
        pltpu.sync_copy(edges_hbm.at[cbase], idx_v.at[0])
        pltpu.async_copy(h_hbm.at[idx_v.at[0, 0]], rows_v.at[0], gsem.at[0])
        pltpu.async_copy(edges_hbm.at[cbase + 1], idx_v.at[1], isem)

        def outer(ci, _):
            for p in range(2):
                c = ci * 2 + p
                @pl.when(c + 1 < cpt)
                def _():
                    pltpu.make_async_copy(edges_hbm.at[cbase],
                                          idx_v.at[1 - p], isem).wait()
                    pltpu.async_copy(h_hbm.at[idx_v.at[1 - p, 0]],
                                     rows_v.at[1 - p], gsem.at[1 - p])
                pltpu.make_async_copy(h_hbm.at[idx_v.at[p, 0]],
                                      rows_v.at[p], gsem.at[p]).wait()
                pltpu.sync_copy(rows_v.at[p], acc_sh.at[idx_v.at[p, 1]],
                                add=True)
                @pl.when(c + 2 < cpt)
                def _():
                    pltpu.async_copy(edges_hbm.at[cbase + c + 2],
                                     idx_v.at[p], isem)
            return ()
        lax.fori_loop(0, cpt // 2, outer, ())

        plsc.subcore_barrier()
        pltpu.sync_copy(acc_sh.at[pl.ds(rbase, rows_per_tile)],
                        out_hbm.at[pl.ds(cid * n_pad + rbase, rows_per_tile)])

    return spmm_kernel


def _tc_matmul(x_pad, w1, n_pad, d):
    blk = 256
    grid = n_pad // blk

    def body(x_ref, w_ref, out_ref):
        out_ref[...] = jnp.dot(x_ref[...], w_ref[...],
                               preferred_element_type=jnp.float32)

    return pl.pallas_call(
        body,
        grid=(grid,),
        in_specs=[
            pl.BlockSpec((blk, d), lambda i: (i, 0)),
            pl.BlockSpec((d, d), lambda i: (0, 0)),
        ],
        out_specs=pl.BlockSpec((blk, d), lambda i: (i, 0)),
        out_shape=jax.ShapeDtypeStruct((n_pad, d), jnp.float32),
    )(x_pad, w1)


def _tc_first(hist3, xw, n_pad, d):
    blk = 256
    grid = n_pad // blk

    def body(hist_ref, xw_ref, h_ref, dis_ref):
        deg = hist_ref[0, :, 0:1] + hist_ref[1, :, 0:1] + 1.0
        dis = lax.rsqrt(deg)
        h_ref[...] = xw_ref[...] * dis
        dis_ref[...] = jnp.broadcast_to(dis, (blk, d))

    return pl.pallas_call(
        body,
        grid=(grid,),
        in_specs=[
            pl.BlockSpec((2, blk, 16), lambda i: (0, i, 0)),
            pl.BlockSpec((blk, d), lambda i: (i, 0)),
        ],
        out_specs=[
            pl.BlockSpec((blk, d), lambda i: (i, 0)),
            pl.BlockSpec((blk, d), lambda i: (i, 0)),
        ],
        out_shape=[
            jax.ShapeDtypeStruct((n_pad, d), jnp.float32),
            jax.ShapeDtypeStruct((n_pad, d), jnp.float32),
        ],
    )(hist3, xw)


def _tc_mid(acc, dis_col, b1r, w2, n_pad, d):
    blk = 256
    grid = n_pad // blk

    def body(a0_ref, a1_ref, dis_ref, b_ref, w_ref, out_ref):
        dis = dis_ref[...]
        agg = (a0_ref[...] + a1_ref[...]) * dis
        h = jnp.maximum(agg + b_ref[...], 0.0)
        hw = jnp.dot(h, w_ref[...], preferred_element_type=jnp.float32)
        out_ref[...] = hw * dis

    return pl.pallas_call(
        body,
        grid=(grid,),
        in_specs=[
            pl.BlockSpec((blk, d), lambda i: (i, 0)),
            pl.BlockSpec((blk, d), lambda i: (i + grid, 0)),
            pl.BlockSpec((blk, d), lambda i: (i, 0)),
            pl.BlockSpec((1, d), lambda i: (0, 0)),
            pl.BlockSpec((d, d), lambda i: (0, 0)),
        ],
        out_specs=pl.BlockSpec((blk, d), lambda i: (i, 0)),
        out_shape=jax.ShapeDtypeStruct((n_pad, d), jnp.float32),
    )(acc, acc, dis_col, b1r, w2)


def _tc_last(acc, dis_col, b2r, n, n_pad, d):
    blk = 400
    grid = n // blk

    def body(a0_ref, a1_ref, dis_ref, b_ref, out_ref):
        z = (a0_ref[...] + a1_ref[...]) * dis_ref[...] + b_ref[...]
        m = jnp.max(z, axis=1, keepdims=True)
        ex = jnp.exp(z - m)
        s = jnp.sum(ex, axis=1, keepdims=True)
        out_ref[...] = z - m - jnp.log(s)

    nblk_off = n_pad // blk

    return pl.pallas_call(
        body,
        grid=(grid,),
        in_specs=[
            pl.BlockSpec((blk, d), lambda i: (i, 0)),
            pl.BlockSpec((blk, d), lambda i: (i + nblk_off, 0)),
            pl.BlockSpec((blk, d), lambda i: (i, 0)),
            pl.BlockSpec((1, d), lambda i: (0, 0)),
        ],
        out_specs=pl.BlockSpec((blk, d), lambda i: (i, 0)),
        out_shape=jax.ShapeDtypeStruct((n, d), jnp.float32),
    )(acc, acc, dis_col, b2r)


def kernel(x, edge_index, W1, b1, W2, b2):
    n, d = x.shape
    e = edge_index.shape[1]

    n_pad = ((n + NS * 256 - 1) // (NS * 256)) * NS * 256
    pad_row = n_pad - 1
    chunks_per_tile = -(-e // (NC * NS * CHUNK))
    chunks_per_tile = ((chunks_per_tile + 1) // 2) * 2
    n_chunks = chunks_per_tile * NC * NS
    e_pad = n_chunks * CHUNK

    src = edge_index[0].astype(jnp.int32)
    dst = edge_index[1].astype(jnp.int32)
    pad = jnp.full((e_pad - e,), pad_row, jnp.int32)
    edges = jnp.stack([
        jnp.concatenate([src, pad]).reshape(n_chunks, CHUNK),
        jnp.concatenate([dst, pad]).reshape(n_chunks, CHUNK),
    ], axis=1)

    x_pad = jnp.concatenate(
        [x, jnp.zeros((n_pad - n, d), jnp.float32)], axis=0)
    zinit = jnp.zeros((n_pad, d), jnp.float32)
    b1r = b1.reshape(1, d)
    b2r = b2.reshape(1, d)

    deg_k = _make_deg_kernel(n_pad, n_chunks)
    spmm_k = _make_spmm_kernel(n_pad, d, n_chunks, 132.0 / 180.0)

    xw = _tc_matmul(x_pad, W1, n_pad, d)
    hist = deg_k(edges).reshape(NC, n_pad, 16)
    h1, dis_col = _tc_first(hist, xw, n_pad, d)
    acc1 = spmm_k(h1, edges, zinit)
    h2 = _tc_mid(acc1, dis_col, b1r, W2, n_pad, d)
    acc2 = spmm_k(h2, edges, zinit)
    return _tc_last(acc2, dis_col, b2r, n, n_pad, d)


# speedup vs baseline: 2.0044x; 1.0382x over previous
import functools

import jax
import jax.numpy as jnp
from jax import lax
from jax.experimental import pallas as pl
from jax.experimental.pallas import tpu as pltpu
from jax.experimental.pallas import tpu_sc as plsc

NC = 2
NS = 16
CHUNK = 112


def _make_deg_kernel(n_pad, n_chunks):
    rows_per_tile = n_pad // NS
    chunks_per_tile = n_chunks // (NC * NS)
    mesh = plsc.VectorSubcoreMesh(core_axis_name="c", subcore_axis_name="s")

    @functools.partial(
        pl.kernel,
        out_type=jax.ShapeDtypeStruct((NC * n_pad, 16), jnp.float32),
        mesh=mesh,
        scratch_types=[
            pltpu.VMEM_SHARED((n_pad, 16), jnp.float32),
            pltpu.VMEM((rows_per_tile, 16), jnp.float32),
            pltpu.VMEM((CHUNK, 16), jnp.float32),
            pltpu.VMEM((2, CHUNK), jnp.int32),
            pltpu.SemaphoreType.DMA,
        ],
    )
    def deg_kernel(edges_hbm, out_hbm, hist_sh, zbuf, ones_v, idx_v, isem):
        cid = lax.axis_index("c")
        sid = lax.axis_index("s")
        wid = sid * NC + cid
        rbase = sid * rows_per_tile

        zrow = jnp.zeros((16,), jnp.float32)
        orow = jnp.ones((16,), jnp.float32)

        def fill_z(i, _):
            zbuf[i, :] = zrow
            return ()
        lax.fori_loop(0, rows_per_tile, fill_z, ())

        def fill_o(i, _):
            ones_v[i, :] = orow
            return ()
        lax.fori_loop(0, CHUNK, fill_o, ())

        pltpu.sync_copy(zbuf, hist_sh.at[pl.ds(rbase, rows_per_tile)])
        plsc.subcore_barrier()

        cpt = chunks_per_tile
        cbase = wid * cpt
        pltpu.sync_copy(edges_hbm.at[cbase, 1], idx_v.at[0])
        pltpu.async_copy(edges_hbm.at[cbase + 1, 1], idx_v.at[1], isem)

        def outer(ci, _):
            for p in range(2):
                c = ci * 2 + p
                @pl.when(c + 1 < cpt)
                def _():
                    pltpu.make_async_copy(edges_hbm.at[cbase, 1],
                                          idx_v.at[1 - p], isem).wait()
                pltpu.sync_copy(ones_v, hist_sh.at[idx_v.at[p]], add=True)
                @pl.when(c + 2 < cpt)
                def _():
                    pltpu.async_copy(edges_hbm.at[cbase + c + 2, 1],
                                     idx_v.at[p], isem)
            return ()
        lax.fori_loop(0, cpt // 2, outer, ())

        plsc.subcore_barrier()
        pltpu.sync_copy(hist_sh.at[pl.ds(rbase, rows_per_tile)],
                        out_hbm.at[pl.ds(cid * n_pad + rbase, rows_per_tile)])

    return deg_kernel


def _make_spmm_kernel(n_pad, d, n_chunks, frac_a):
    rows_per_tile = n_pad // NS
    cpt_pair = n_chunks // NS
    cpt_a = int(round(cpt_pair * frac_a / 2)) * 2
    cpt_b = cpt_pair - cpt_a
    mesh = plsc.VectorSubcoreMesh(core_axis_name="c", subcore_axis_name="s")

    @functools.partial(
        pl.kernel,
        out_type=jax.ShapeDtypeStruct((NC * n_pad, d), jnp.float32),
        mesh=mesh,
        scratch_types=[
            pltpu.VMEM_SHARED((n_pad, d), jnp.float32),
            pltpu.VMEM((2, CHUNK, d), jnp.float32),
            pltpu.VMEM((2, 2, CHUNK), jnp.int32),
            pltpu.SemaphoreType.DMA((2,)),
            pltpu.SemaphoreType.DMA,
        ],
    )
    def spmm_kernel(h_hbm, edges_hbm, z_hbm, out_hbm,
                    acc_sh, rows_v, idx_v, gsem, isem):
        cid = lax.axis_index("c")
        sid = lax.axis_index("s")
        rbase = sid * rows_per_tile
        cpt = jnp.where(cid == 0, cpt_a, cpt_b)

        @pl.when(cid == 0)
        def _():
            pltpu.sync_copy(h_hbm.at[pl.ds(rbase, rows_per_tile)],
                            acc_sh.at[pl.ds(rbase, rows_per_tile)])

        @pl.when(cid == 1)
        def _():
            pltpu.sync_copy(z_hbm.at[pl.ds(rbase, rows_per_tile)],
                            acc_sh.at[pl.ds(rbase, rows_per_tile)])

        plsc.subcore_barrier()

        cbase = sid * cpt_pair + jnp.where(cid == 0, 0, cpt_a)
        pltpu.sync_copy(edges_hbm.at[cbase], idx_v.at[0])
        pltpu.async_copy(h_hbm.at[idx_v.at[0, 0]], rows_v.at[0], gsem.at[0])
        pltpu.async_copy(edges_hbm.at[cbase + 1], idx_v.at[1], isem)

        def outer(ci, _):
            for p in range(2):
                c = ci * 2 + p
                @pl.when(c + 1 < cpt)
                def _():
                    pltpu.make_async_copy(edges_hbm.at[cbase],
                                          idx_v.at[1 - p], isem).wait()
                    pltpu.async_copy(h_hbm.at[idx_v.at[1 - p, 0]],
                                     rows_v.at[1 - p], gsem.at[1 - p])
                pltpu.make_async_copy(h_hbm.at[idx_v.at[p, 0]],
                                      rows_v.at[p], gsem.at[p]).wait()
                pltpu.sync_copy(rows_v.at[p], acc_sh.at[idx_v.at[p, 1]],
                                add=True)
                @pl.when(c + 2 < cpt)
                def _():
                    pltpu.async_copy(edges_hbm.at[cbase + c + 2],
                                     idx_v.at[p], isem)
            return ()
        lax.fori_loop(0, cpt // 2, outer, ())

        plsc.subcore_barrier()
        pltpu.sync_copy(acc_sh.at[pl.ds(rbase, rows_per_tile)],
                        out_hbm.at[pl.ds(cid * n_pad + rbase, rows_per_tile)])

    return spmm_kernel


def _tc_matmul(x_pad, w1, n_pad, d):
    blk = 256
    grid = n_pad // blk

    def body(x_ref, w_ref, out_ref):
        out_ref[...] = jnp.dot(x_ref[...], w_ref[...],
                               preferred_element_type=jnp.float32)

    return pl.pallas_call(
        body,
        grid=(grid,),
        in_specs=[
            pl.BlockSpec((blk, d), lambda i: (i, 0)),
            pl.BlockSpec((d, d), lambda i: (0, 0)),
        ],
        out_specs=pl.BlockSpec((blk, d), lambda i: (i, 0)),
        out_shape=jax.ShapeDtypeStruct((n_pad, d), jnp.float32),
    )(x_pad, w1)


def _tc_first(hist3, xw, n_pad, d):
    blk = 256
    grid = n_pad // blk

    def body(hist_ref, xw_ref, h_ref, dis_ref):
        deg = hist_ref[0, :, 0:1] + hist_ref[1, :, 0:1] + 1.0
        dis = lax.rsqrt(deg)
        h_ref[...] = xw_ref[...] * dis
        dis_ref[...] = jnp.broadcast_to(dis, (blk, d))

    return pl.pallas_call(
        body,
        grid=(grid,),
        in_specs=[
            pl.BlockSpec((2, blk, 16), lambda i: (0, i, 0)),
            pl.BlockSpec((blk, d), lambda i: (i, 0)),
        ],
        out_specs=[
            pl.BlockSpec((blk, d), lambda i: (i, 0)),
            pl.BlockSpec((blk, d), lambda i: (i, 0)),
        ],
        out_shape=[
            jax.ShapeDtypeStruct((n_pad, d), jnp.float32),
            jax.ShapeDtypeStruct((n_pad, d), jnp.float32),
        ],
    )(hist3, xw)


def _tc_mid(acc, dis_col, b1r, w2, n_pad, d):
    blk = 256
    grid = n_pad // blk

    def body(a0_ref, a1_ref, dis_ref, b_ref, w_ref, out_ref):
        dis = dis_ref[...]
        agg = (a0_ref[...] + a1_ref[...]) * dis
        h = jnp.maximum(agg + b_ref[...], 0.0)
        hw = jnp.dot(h, w_ref[...], preferred_element_type=jnp.float32)
        out_ref[...] = hw * dis

    return pl.pallas_call(
        body,
        grid=(grid,),
        in_specs=[
            pl.BlockSpec((blk, d), lambda i: (i, 0)),
            pl.BlockSpec((blk, d), lambda i: (i + grid, 0)),
            pl.BlockSpec((blk, d), lambda i: (i, 0)),
            pl.BlockSpec((1, d), lambda i: (0, 0)),
            pl.BlockSpec((d, d), lambda i: (0, 0)),
        ],
        out_specs=pl.BlockSpec((blk, d), lambda i: (i, 0)),
        out_shape=jax.ShapeDtypeStruct((n_pad, d), jnp.float32),
    )(acc, acc, dis_col, b1r, w2)


def _tc_last(acc, dis_col, b2r, n, n_pad, d):
    blk = 400
    grid = n // blk

    def body(a0_ref, a1_ref, dis_ref, b_ref, out_ref):
        z = (a0_ref[...] + a1_ref[...]) * dis_ref[...] + b_ref[...]
        m = jnp.max(z, axis=1, keepdims=True)
        ex = jnp.exp(z - m)
        s = jnp.sum(ex, axis=1, keepdims=True)
        out_ref[...] = z - m - jnp.log(s)

    nblk_off = n_pad // blk

    return pl.pallas_call(
        body,
        grid=(grid,),
        in_specs=[
            pl.BlockSpec((blk, d), lambda i: (i, 0)),
            pl.BlockSpec((blk, d), lambda i: (i + nblk_off, 0)),
            pl.BlockSpec((blk, d), lambda i: (i, 0)),
            pl.BlockSpec((1, d), lambda i: (0, 0)),
        ],
        out_specs=pl.BlockSpec((blk, d), lambda i: (i, 0)),
        out_shape=jax.ShapeDtypeStruct((n, d), jnp.float32),
    )(acc, acc, dis_col, b2r)


def kernel(x, edge_index, W1, b1, W2, b2):
    n, d = x.shape
    e = edge_index.shape[1]

    n_pad = ((n + NS * 256 - 1) // (NS * 256)) * NS * 256
    if n_pad == n:
        n_pad += NS * 256
    chunks_per_tile = -(-e // (NC * NS * CHUNK))
    chunks_per_tile = ((chunks_per_tile + 1) // 2) * 2
    n_chunks = chunks_per_tile * NC * NS
    e_pad = n_chunks * CHUNK

    src = edge_index[0].astype(jnp.int32)
    dst = edge_index[1].astype(jnp.int32)
    n_spare = n_pad - n
    pad = n + jax.lax.rem(jnp.arange(e_pad - e, dtype=jnp.int32),
                          jnp.int32(n_spare))
    edges = jnp.stack([
        jnp.concatenate([src, pad]).reshape(n_chunks, CHUNK),
        jnp.concatenate([dst, pad]).reshape(n_chunks, CHUNK),
    ], axis=1)

    x_pad = jnp.concatenate(
        [x, jnp.zeros((n_pad - n, d), jnp.float32)], axis=0)
    zinit = jnp.zeros((n_pad, d), jnp.float32)
    b1r = b1.reshape(1, d)
    b2r = b2.reshape(1, d)

    deg_k = _make_deg_kernel(n_pad, n_chunks)
    spmm_k = _make_spmm_kernel(n_pad, d, n_chunks, 132.0 / 180.0)

    xw = _tc_matmul(x_pad, W1, n_pad, d)
    hist = deg_k(edges).reshape(NC, n_pad, 16)
    h1, dis_col = _tc_first(hist, xw, n_pad, d)
    acc1 = spmm_k(h1, edges, zinit)
    h2 = _tc_mid(acc1, dis_col, b1r, W2, n_pad, d)
    acc2 = spmm_k(h2, edges, zinit)
    return _tc_last(acc2, dis_col, b2r, n, n_pad, d)
